# baseline (speedup 1.0000x reference)
import jax
import jax.numpy as jnp
from jax.experimental import pallas as pl
from jax.experimental.pallas import tpu as pltpu

_B = 4096
_H = 1024
_BB = 256
_NB = _B // _BB
_NPAIR = (_NB - 2) // 2


def _dot_t(a, w):
    return jax.lax.dot_general(
        a, w, (((1,), (1,)), ((), ())), preferred_element_type=jnp.float32
    )


def _lstm_body(x_hbm, hx_hbm, cx_hbm, w0_hbm, w1_hbm, w2_hbm, w3_hbm, w4_hbm,
               w5_hbm, w6_hbm, bxt, btf, bcf, btu, bcu, bth, bch,
               hy_hbm, cy_hbm,
               wvm, xbuf, hxbuf, cxbuf, hybuf, cybuf,
               wsem, xsem, hxsem, cxsem, hysem, cysem):
    w_hbm = (w0_hbm, w1_hbm, w2_hbm, w3_hbm, w4_hbm, w5_hbm, w6_hbm)

    def _prefetch(slot, i):
        pltpu.make_async_copy(
            x_hbm.at[pl.ds(i * _BB, _BB), :], xbuf.at[slot],
            xsem.at[slot]).start()
        pltpu.make_async_copy(
            hx_hbm.at[pl.ds(i * _BB, _BB), :], hxbuf.at[slot],
            hxsem.at[slot]).start()
        pltpu.make_async_copy(
            cx_hbm.at[pl.ds(i * _BB, _BB), :], cxbuf.at[slot],
            cxsem.at[slot]).start()

    _prefetch(0, 0)
    pltpu.make_async_copy(w_hbm[0], wvm.at[0], wsem.at[0]).start()
    _prefetch(1, 1)
    for j in range(1, 7):
        pltpu.make_async_copy(w_hbm[j], wvm.at[j], wsem.at[j]).start()

    def _wait_w(j):
        pltpu.make_async_copy(w_hbm[j], wvm.at[j], wsem.at[j]).wait()

    def _wait_acts(slot):
        pltpu.make_async_copy(xbuf.at[slot], xbuf.at[slot], xsem.at[slot]).wait()
        pltpu.make_async_copy(hxbuf.at[slot], hxbuf.at[slot], hxsem.at[slot]).wait()
        pltpu.make_async_copy(cxbuf.at[slot], cxbuf.at[slot], cxsem.at[slot]).wait()

    def _wait_outs(slot, i):
        pltpu.make_async_copy(
            hybuf.at[slot], hy_hbm.at[pl.ds(i * _BB, _BB), :],
            hysem.at[slot]).wait()
        pltpu.make_async_copy(
            cybuf.at[slot], cy_hbm.at[pl.ds(i * _BB, _BB), :],
            cysem.at[slot]).wait()

    def _compute(x, hx, cx):
        t = jnp.tanh(_dot_t(x, wvm[0]) + bxt[...]) + hx
        f = jax.nn.sigmoid(
            _dot_t(t, wvm[1]) + _dot_t(cx, wvm[2]) + (btf[...] + bcf[...])
        )
        u = jax.nn.sigmoid(
            _dot_t(t, wvm[3]) + _dot_t(cx, wvm[4]) + (btu[...] + bcu[...])
        ) * t
        cy = jnp.tanh(f * cx + u)
        hy = jnp.tanh(
            jax.nn.sigmoid(
                _dot_t(t, wvm[5]) + _dot_t(cy, wvm[6]) + (bth[...] + bch[...])
            ) * cy
        )
        return hy, cy

    def _emit_out(slot, i, hy, cy):
        hybuf[slot] = hy
        cybuf[slot] = cy
        pltpu.make_async_copy(
            hybuf.at[slot], hy_hbm.at[pl.ds(i * _BB, _BB), :],
            hysem.at[slot]).start()
        pltpu.make_async_copy(
            cybuf.at[slot], cy_hbm.at[pl.ds(i * _BB, _BB), :],
            cysem.at[slot]).start()

    _wait_acts(0)
    x0, hx0, cx0 = xbuf[0], hxbuf[0], cxbuf[0]
    _wait_w(0)
    t0 = jnp.tanh(_dot_t(x0, wvm[0]) + bxt[...]) + hx0
    _wait_w(1)
    _wait_w(2)
    f0 = jax.nn.sigmoid(
        _dot_t(t0, wvm[1]) + _dot_t(cx0, wvm[2]) + (btf[...] + bcf[...])
    )
    _wait_w(3)
    _wait_w(4)
    u0 = jax.nn.sigmoid(
        _dot_t(t0, wvm[3]) + _dot_t(cx0, wvm[4]) + (btu[...] + bcu[...])
    ) * t0
    cy0 = jnp.tanh(f0 * cx0 + u0)
    _wait_w(5)
    _wait_w(6)
    hy0 = jnp.tanh(
        jax.nn.sigmoid(
            _dot_t(t0, wvm[5]) + _dot_t(cy0, wvm[6]) + (bth[...] + bch[...])
        ) * cy0
    )
    _emit_out(0, 0, hy0, cy0)
    _prefetch(2, 2)

    _wait_acts(1)
    hy1, cy1 = _compute(xbuf[1], hxbuf[1], cxbuf[1])
    _emit_out(1, 1, hy1, cy1)
    _prefetch(3, 3)

    def step(it, carry):
        j0 = 2 + 2 * it
        cur = 2 * jax.lax.rem(it + 1, 2)
        nxt = 2 - cur
        _prefetch(nxt, jnp.minimum(j0 + 2, _NB - 2))
        _prefetch(nxt + 1, jnp.minimum(j0 + 3, _NB - 1))
        _wait_acts(cur)
        _wait_acts(cur + 1)
        _wait_outs(0, j0)
        _wait_outs(1, j0 + 1)
        hy_a, cy_a = _compute(xbuf[cur], hxbuf[cur], cxbuf[cur])
        hy_b, cy_b = _compute(xbuf[cur + 1], hxbuf[cur + 1], cxbuf[cur + 1])
        _emit_out(0, j0, hy_a, cy_a)
        _emit_out(1, j0 + 1, hy_b, cy_b)
        return carry

    jax.lax.fori_loop(0, _NPAIR, step, 0)

    _wait_outs(0, _NB - 2)
    _wait_outs(1, _NB - 1)
    nxt_last = 2 - 2 * (_NPAIR % 2)
    _wait_acts(nxt_last)
    _wait_acts(nxt_last + 1)


def kernel(x, hx, cx, W_xt, W_tf, W_cf, W_tu, W_cu, W_th, W_ch,
           b_xt, b_tf, b_cf, b_tu, b_cu, b_th, b_ch):
    any_spec = pl.BlockSpec(memory_space=pl.MemorySpace.ANY)
    vmem_spec = pl.BlockSpec(memory_space=pltpu.VMEM)
    out = pl.pallas_call(
        _lstm_body,
        in_specs=[any_spec] * 10 + [vmem_spec] * 7,
        out_specs=[any_spec, any_spec],
        out_shape=[
            jax.ShapeDtypeStruct((_B, _H), jnp.float32),
            jax.ShapeDtypeStruct((_B, _H), jnp.float32),
        ],
        scratch_shapes=[
            pltpu.VMEM((7, _H, _H), jnp.float32),
            pltpu.VMEM((4, _BB, _H), jnp.float32),
            pltpu.VMEM((4, _BB, _H), jnp.float32),
            pltpu.VMEM((4, _BB, _H), jnp.float32),
            pltpu.VMEM((2, _BB, _H), jnp.float32),
            pltpu.VMEM((2, _BB, _H), jnp.float32),
            pltpu.SemaphoreType.DMA((7,)),
            pltpu.SemaphoreType.DMA((4,)),
            pltpu.SemaphoreType.DMA((4,)),
            pltpu.SemaphoreType.DMA((4,)),
            pltpu.SemaphoreType.DMA((2,)),
            pltpu.SemaphoreType.DMA((2,)),
        ],
        compiler_params=pltpu.CompilerParams(
            vmem_limit_bytes=65024 * 1024,
        ),
        name="fused_lstm_cell_manual",
    )(x, hx, cx, W_xt, W_tf, W_cf, W_tu, W_cu, W_th, W_ch,
      b_xt.reshape(1, _H), b_tf.reshape(1, _H), b_cf.reshape(1, _H),
      b_tu.reshape(1, _H), b_cu.reshape(1, _H), b_th.reshape(1, _H),
      b_ch.reshape(1, _H))
    return (out[0], out[1])


# speedup vs baseline: 1.0526x; 1.0526x over previous
import jax
import jax.numpy as jnp
from jax.experimental import pallas as pl
from jax.experimental.pallas import tpu as pltpu

_B = 4096
_H = 1024
_BB = 256
_NB = _B // _BB
_NPAIR = (_NB - 2) // 2


def _dot_t(a, w):
    return jax.lax.dot_general(
        a, w, (((1,), (1,)), ((), ())), preferred_element_type=jnp.float32
    )


def _lstm_body(x_hbm, hx_hbm, cx_hbm, w0_hbm, w1_hbm, w2_hbm, w3_hbm, w4_hbm,
               w5_hbm, w6_hbm, bxt, btf, bcf, btu, bcu, bth, bch,
               hy_hbm, cy_hbm,
               wvm, xbuf, hxbuf, cxbuf, hybuf, cybuf,
               wsem, xsem, hxsem, cxsem, hysem, cysem):
    w_hbm = (w0_hbm, w1_hbm, w2_hbm, w3_hbm, w4_hbm, w5_hbm, w6_hbm)

    def _prefetch(slot, i):
        pltpu.make_async_copy(
            x_hbm.at[pl.ds(i * _BB, _BB), :], xbuf.at[slot],
            xsem.at[slot]).start()
        pltpu.make_async_copy(
            hx_hbm.at[pl.ds(i * _BB, _BB), :], hxbuf.at[slot],
            hxsem.at[slot]).start()
        pltpu.make_async_copy(
            cx_hbm.at[pl.ds(i * _BB, _BB), :], cxbuf.at[slot],
            cxsem.at[slot]).start()

    _prefetch(0, 0)
    pltpu.make_async_copy(w_hbm[0], wvm.at[0], wsem.at[0]).start()
    _prefetch(1, 1)
    for j in range(1, 7):
        pltpu.make_async_copy(w_hbm[j], wvm.at[j], wsem.at[j]).start()

    def _wait_w(j):
        pltpu.make_async_copy(w_hbm[j], wvm.at[j], wsem.at[j]).wait()

    def _wait_acts(slot):
        pltpu.make_async_copy(xbuf.at[slot], xbuf.at[slot], xsem.at[slot]).wait()
        pltpu.make_async_copy(hxbuf.at[slot], hxbuf.at[slot], hxsem.at[slot]).wait()
        pltpu.make_async_copy(cxbuf.at[slot], cxbuf.at[slot], cxsem.at[slot]).wait()

    def _wait_outs(slot, i):
        pltpu.make_async_copy(
            hybuf.at[slot], hy_hbm.at[pl.ds(i * _BB, _BB), :],
            hysem.at[slot]).wait()
        pltpu.make_async_copy(
            cybuf.at[slot], cy_hbm.at[pl.ds(i * _BB, _BB), :],
            cysem.at[slot]).wait()

    def _compute(x, hx, cx):
        t = jnp.tanh(_dot_t(x, wvm[0]) + bxt[...]) + hx
        f = jax.nn.sigmoid(
            _dot_t(t, wvm[1]) + _dot_t(cx, wvm[2]) + (btf[...] + bcf[...])
        )
        u = jax.nn.sigmoid(
            _dot_t(t, wvm[3]) + _dot_t(cx, wvm[4]) + (btu[...] + bcu[...])
        ) * t
        cy = jnp.tanh(f * cx + u)
        hy = jnp.tanh(
            jax.nn.sigmoid(
                _dot_t(t, wvm[5]) + _dot_t(cy, wvm[6]) + (bth[...] + bch[...])
            ) * cy
        )
        return hy, cy

    def _emit_out(slot, i, hy, cy):
        hybuf[slot] = hy
        cybuf[slot] = cy
        pltpu.make_async_copy(
            hybuf.at[slot], hy_hbm.at[pl.ds(i * _BB, _BB), :],
            hysem.at[slot]).start()
        pltpu.make_async_copy(
            cybuf.at[slot], cy_hbm.at[pl.ds(i * _BB, _BB), :],
            cysem.at[slot]).start()

    _wait_acts(0)
    x0, hx0, cx0 = xbuf[0], hxbuf[0], cxbuf[0]
    _wait_w(0)
    t0 = jnp.tanh(_dot_t(x0, wvm[0]) + bxt[...]) + hx0
    _wait_w(1)
    _wait_w(2)
    f0 = jax.nn.sigmoid(
        _dot_t(t0, wvm[1]) + _dot_t(cx0, wvm[2]) + (btf[...] + bcf[...])
    )
    _wait_w(3)
    _wait_w(4)
    u0 = jax.nn.sigmoid(
        _dot_t(t0, wvm[3]) + _dot_t(cx0, wvm[4]) + (btu[...] + bcu[...])
    ) * t0
    cy0 = jnp.tanh(f0 * cx0 + u0)
    _wait_w(5)
    _wait_w(6)
    hy0 = jnp.tanh(
        jax.nn.sigmoid(
            _dot_t(t0, wvm[5]) + _dot_t(cy0, wvm[6]) + (bth[...] + bch[...])
        ) * cy0
    )
    _emit_out(0, 0, hy0, cy0)
    _prefetch(2, 2)

    _wait_acts(1)
    hy1, cy1 = _compute(xbuf[1], hxbuf[1], cxbuf[1])
    _emit_out(1, 1, hy1, cy1)
    _prefetch(3, 3)

    def step(it, carry):
        j0 = 2 + 2 * it
        cur = 2 * jax.lax.rem(it + 1, 2)
        nxt = 2 - cur
        _prefetch(nxt, jnp.minimum(j0 + 2, _NB - 2))
        _prefetch(nxt + 1, jnp.minimum(j0 + 3, _NB - 1))
        _wait_acts(cur)
        _wait_acts(cur + 1)
        hy_a, cy_a = _compute(xbuf[cur], hxbuf[cur], cxbuf[cur])
        hy_b, cy_b = _compute(xbuf[cur + 1], hxbuf[cur + 1], cxbuf[cur + 1])
        _wait_outs(0, j0)
        _wait_outs(1, j0 + 1)
        _emit_out(0, j0, hy_a, cy_a)
        _emit_out(1, j0 + 1, hy_b, cy_b)
        return carry

    jax.lax.fori_loop(0, _NPAIR, step, 0)

    _wait_outs(0, _NB - 2)
    _wait_outs(1, _NB - 1)
    nxt_last = 2 - 2 * (_NPAIR % 2)
    _wait_acts(nxt_last)
    _wait_acts(nxt_last + 1)


def kernel(x, hx, cx, W_xt, W_tf, W_cf, W_tu, W_cu, W_th, W_ch,
           b_xt, b_tf, b_cf, b_tu, b_cu, b_th, b_ch):
    any_spec = pl.BlockSpec(memory_space=pl.MemorySpace.ANY)
    vmem_spec = pl.BlockSpec(memory_space=pltpu.VMEM)
    out = pl.pallas_call(
        _lstm_body,
        in_specs=[any_spec] * 10 + [vmem_spec] * 7,
        out_specs=[any_spec, any_spec],
        out_shape=[
            jax.ShapeDtypeStruct((_B, _H), jnp.float32),
            jax.ShapeDtypeStruct((_B, _H), jnp.float32),
        ],
        scratch_shapes=[
            pltpu.VMEM((7, _H, _H), jnp.float32),
            pltpu.VMEM((4, _BB, _H), jnp.float32),
            pltpu.VMEM((4, _BB, _H), jnp.float32),
            pltpu.VMEM((4, _BB, _H), jnp.float32),
            pltpu.VMEM((2, _BB, _H), jnp.float32),
            pltpu.VMEM((2, _BB, _H), jnp.float32),
            pltpu.SemaphoreType.DMA((7,)),
            pltpu.SemaphoreType.DMA((4,)),
            pltpu.SemaphoreType.DMA((4,)),
            pltpu.SemaphoreType.DMA((4,)),
            pltpu.SemaphoreType.DMA((2,)),
            pltpu.SemaphoreType.DMA((2,)),
        ],
        compiler_params=pltpu.CompilerParams(
            vmem_limit_bytes=65024 * 1024,
        ),
        name="fused_lstm_cell_manual",
    )(x, hx, cx, W_xt, W_tf, W_cf, W_tu, W_cu, W_th, W_ch,
      b_xt.reshape(1, _H), b_tf.reshape(1, _H), b_cf.reshape(1, _H),
      b_tu.reshape(1, _H), b_cu.reshape(1, _H), b_th.reshape(1, _H),
      b_ch.reshape(1, _H))
    return (out[0], out[1])


# speedup vs baseline: 1.0760x; 1.0222x over previous
import jax
import jax.numpy as jnp
from jax.experimental import pallas as pl
from jax.experimental.pallas import tpu as pltpu

_B = 4096
_H = 1024
_BB = 256
_NB = _B // _BB
_NPAIR = (_NB - 2) // 2


def _dot_t(a, w):
    return jax.lax.dot_general(
        a, w, (((1,), (1,)), ((), ())), preferred_element_type=jnp.float32
    )


def _sigmoid(z):
    return 0.5 * jnp.tanh(0.5 * z) + 0.5


def _lstm_body(x_hbm, hx_hbm, cx_hbm, w0_hbm, w1_hbm, w2_hbm, w3_hbm, w4_hbm,
               w5_hbm, w6_hbm, bxt, btf, bcf, btu, bcu, bth, bch,
               hy_hbm, cy_hbm,
               wvm, xbuf, hxbuf, cxbuf, hybuf, cybuf,
               wsem, xsem, hxsem, cxsem, hysem, cysem):
    w_hbm = (w0_hbm, w1_hbm, w2_hbm, w3_hbm, w4_hbm, w5_hbm, w6_hbm)

    def _prefetch(slot, i):
        pltpu.make_async_copy(
            x_hbm.at[pl.ds(i * _BB, _BB), :], xbuf.at[slot],
            xsem.at[slot]).start()
        pltpu.make_async_copy(
            hx_hbm.at[pl.ds(i * _BB, _BB), :], hxbuf.at[slot],
            hxsem.at[slot]).start()
        pltpu.make_async_copy(
            cx_hbm.at[pl.ds(i * _BB, _BB), :], cxbuf.at[slot],
            cxsem.at[slot]).start()

    _prefetch(0, 0)
    pltpu.make_async_copy(w_hbm[0], wvm.at[0], wsem.at[0]).start()
    _prefetch(1, 1)
    for j in range(1, 7):
        pltpu.make_async_copy(w_hbm[j], wvm.at[j], wsem.at[j]).start()

    def _wait_w(j):
        pltpu.make_async_copy(w_hbm[j], wvm.at[j], wsem.at[j]).wait()

    def _wait_acts(slot):
        pltpu.make_async_copy(xbuf.at[slot], xbuf.at[slot], xsem.at[slot]).wait()
        pltpu.make_async_copy(hxbuf.at[slot], hxbuf.at[slot], hxsem.at[slot]).wait()
        pltpu.make_async_copy(cxbuf.at[slot], cxbuf.at[slot], cxsem.at[slot]).wait()

    def _wait_outs(slot, i):
        pltpu.make_async_copy(
            hybuf.at[slot], hy_hbm.at[pl.ds(i * _BB, _BB), :],
            hysem.at[slot]).wait()
        pltpu.make_async_copy(
            cybuf.at[slot], cy_hbm.at[pl.ds(i * _BB, _BB), :],
            cysem.at[slot]).wait()

    def _compute(x, hx, cx):
        t = jnp.tanh(_dot_t(x, wvm[0]) + bxt[...]) + hx
        f = _sigmoid(
            _dot_t(t, wvm[1]) + _dot_t(cx, wvm[2]) + (btf[...] + bcf[...])
        )
        u = _sigmoid(
            _dot_t(t, wvm[3]) + _dot_t(cx, wvm[4]) + (btu[...] + bcu[...])
        ) * t
        cy = jnp.tanh(f * cx + u)
        hy = jnp.tanh(
            _sigmoid(
                _dot_t(t, wvm[5]) + _dot_t(cy, wvm[6]) + (bth[...] + bch[...])
            ) * cy
        )
        return hy, cy

    def _emit_out(slot, i, hy, cy):
        hybuf[slot] = hy
        cybuf[slot] = cy
        pltpu.make_async_copy(
            hybuf.at[slot], hy_hbm.at[pl.ds(i * _BB, _BB), :],
            hysem.at[slot]).start()
        pltpu.make_async_copy(
            cybuf.at[slot], cy_hbm.at[pl.ds(i * _BB, _BB), :],
            cysem.at[slot]).start()

    _wait_acts(0)
    x0, hx0, cx0 = xbuf[0], hxbuf[0], cxbuf[0]
    _wait_w(0)
    t0 = jnp.tanh(_dot_t(x0, wvm[0]) + bxt[...]) + hx0
    _wait_w(1)
    _wait_w(2)
    f0 = _sigmoid(
        _dot_t(t0, wvm[1]) + _dot_t(cx0, wvm[2]) + (btf[...] + bcf[...])
    )
    _wait_w(3)
    _wait_w(4)
    u0 = _sigmoid(
        _dot_t(t0, wvm[3]) + _dot_t(cx0, wvm[4]) + (btu[...] + bcu[...])
    ) * t0
    cy0 = jnp.tanh(f0 * cx0 + u0)
    _wait_w(5)
    _wait_w(6)
    hy0 = jnp.tanh(
        _sigmoid(
            _dot_t(t0, wvm[5]) + _dot_t(cy0, wvm[6]) + (bth[...] + bch[...])
        ) * cy0
    )
    _emit_out(0, 0, hy0, cy0)
    _prefetch(2, 2)

    _wait_acts(1)
    hy1, cy1 = _compute(xbuf[1], hxbuf[1], cxbuf[1])
    _emit_out(1, 1, hy1, cy1)
    _prefetch(3, 3)

    def step(it, carry):
        j0 = 2 + 2 * it
        cur = 2 * jax.lax.rem(it + 1, 2)
        nxt = 2 - cur
        _prefetch(nxt, jnp.minimum(j0 + 2, _NB - 2))
        _prefetch(nxt + 1, jnp.minimum(j0 + 3, _NB - 1))
        _wait_acts(cur)
        _wait_acts(cur + 1)
        hy_a, cy_a = _compute(xbuf[cur], hxbuf[cur], cxbuf[cur])
        hy_b, cy_b = _compute(xbuf[cur + 1], hxbuf[cur + 1], cxbuf[cur + 1])
        _wait_outs(0, j0)
        _wait_outs(1, j0 + 1)
        _emit_out(0, j0, hy_a, cy_a)
        _emit_out(1, j0 + 1, hy_b, cy_b)
        return carry

    jax.lax.fori_loop(0, _NPAIR, step, 0)

    _wait_outs(0, _NB - 2)
    _wait_outs(1, _NB - 1)
    nxt_last = 2 - 2 * (_NPAIR % 2)
    _wait_acts(nxt_last)
    _wait_acts(nxt_last + 1)


def kernel(x, hx, cx, W_xt, W_tf, W_cf, W_tu, W_cu, W_th, W_ch,
           b_xt, b_tf, b_cf, b_tu, b_cu, b_th, b_ch):
    any_spec = pl.BlockSpec(memory_space=pl.MemorySpace.ANY)
    vmem_spec = pl.BlockSpec(memory_space=pltpu.VMEM)
    out = pl.pallas_call(
        _lstm_body,
        in_specs=[any_spec] * 10 + [vmem_spec] * 7,
        out_specs=[any_spec, any_spec],
        out_shape=[
            jax.ShapeDtypeStruct((_B, _H), jnp.float32),
            jax.ShapeDtypeStruct((_B, _H), jnp.float32),
        ],
        scratch_shapes=[
            pltpu.VMEM((7, _H, _H), jnp.float32),
            pltpu.VMEM((4, _BB, _H), jnp.float32),
            pltpu.VMEM((4, _BB, _H), jnp.float32),
            pltpu.VMEM((4, _BB, _H), jnp.float32),
            pltpu.VMEM((2, _BB, _H), jnp.float32),
            pltpu.VMEM((2, _BB, _H), jnp.float32),
            pltpu.SemaphoreType.DMA((7,)),
            pltpu.SemaphoreType.DMA((4,)),
            pltpu.SemaphoreType.DMA((4,)),
            pltpu.SemaphoreType.DMA((4,)),
            pltpu.SemaphoreType.DMA((2,)),
            pltpu.SemaphoreType.DMA((2,)),
        ],
        compiler_params=pltpu.CompilerParams(
            vmem_limit_bytes=65024 * 1024,
        ),
        name="fused_lstm_cell_manual",
    )(x, hx, cx, W_xt, W_tf, W_cf, W_tu, W_cu, W_th, W_ch,
      b_xt.reshape(1, _H), b_tf.reshape(1, _H), b_cf.reshape(1, _H),
      b_tu.reshape(1, _H), b_cu.reshape(1, _H), b_th.reshape(1, _H),
      b_ch.reshape(1, _H))
    return (out[0], out[1])
